# Initial kernel scaffold; baseline (speedup 1.0000x reference)
import jax
import jax.numpy as jnp
from jax.experimental import pallas as pl


def kernel(input_ids, table):
    raise NotImplementedError("write your pallas kernel here")


# speedup vs baseline: 1.5290x; 1.5290x over previous
import functools

import jax
import jax.numpy as jnp
from jax import lax
from jax.experimental import pallas as pl
from jax.experimental.pallas import tpu as pltpu
from jax.experimental.pallas import tpu_sc as plsc

VOCAB = 100000
D_MODEL = 4096
BATCH = 4
SEQ = 8192

_B = BATCH * SEQ

_info = plsc.get_sparse_core_info()
_NC, _NS = _info.num_cores, _info.num_subcores
_NW = _NC * _NS
_B_PER_W = _B // _NW
_CHUNK = 8
_NITER = _B_PER_W // _CHUNK


def _sc_embed(ids_flat, table):
    mesh = plsc.VectorSubcoreMesh(core_axis_name="c", subcore_axis_name="s")

    @functools.partial(
        pl.kernel,
        mesh=mesh,
        out_type=jax.ShapeDtypeStruct((_B, D_MODEL), jnp.float32),
        scratch_types=[
            pltpu.VMEM((_B_PER_W,), jnp.int32),
            pltpu.VMEM((_CHUNK, D_MODEL), jnp.float32),
            pltpu.SemaphoreType.DMA,
        ],
    )
    def k(ids_hbm, table_hbm, out_hbm, idx_v, rows_v, gsem):
        wid = lax.axis_index("s") * _NC + lax.axis_index("c")
        base = wid * _B_PER_W
        pltpu.sync_copy(ids_hbm.at[pl.ds(base, _B_PER_W)], idx_v)

        def body(c, carry):
            off = pl.multiple_of(c * _CHUNK, 8)
            pltpu.async_copy(
                table_hbm.at[idx_v.at[pl.ds(off, _CHUNK)]], rows_v, gsem
            ).wait()
            pltpu.sync_copy(rows_v, out_hbm.at[pl.ds(base + off, _CHUNK)])
            return carry

        lax.fori_loop(0, _NITER, body, 0)

    return k(ids_flat, table)


def kernel(input_ids, table):
    ids_flat = input_ids.reshape(-1).astype(jnp.int32)
    out = _sc_embed(ids_flat, table)
    return out.reshape(BATCH, SEQ, D_MODEL)


# speedup vs baseline: 1.8756x; 1.2267x over previous
import functools

import jax
import jax.numpy as jnp
from jax import lax
from jax.experimental import pallas as pl
from jax.experimental.pallas import tpu as pltpu
from jax.experimental.pallas import tpu_sc as plsc

VOCAB = 100000
D_MODEL = 4096
BATCH = 4
SEQ = 8192

_B = BATCH * SEQ

_info = plsc.get_sparse_core_info()
_NC, _NS = _info.num_cores, _info.num_subcores
_NW = _NC * _NS
_B_PER_W = _B // _NW
_CHUNK = 8
_NITER = _B_PER_W // _CHUNK
_NBUF = 2


def _sc_embed(ids_flat, table):
    mesh = plsc.VectorSubcoreMesh(core_axis_name="c", subcore_axis_name="s")

    @functools.partial(
        pl.kernel,
        mesh=mesh,
        out_type=jax.ShapeDtypeStruct((_B, D_MODEL), jnp.float32),
        scratch_types=[
            pltpu.VMEM((_B_PER_W,), jnp.int32),
            pltpu.VMEM((_CHUNK, D_MODEL), jnp.float32),
            pltpu.VMEM((_CHUNK, D_MODEL), jnp.float32),
            pltpu.SemaphoreType.DMA,
            pltpu.SemaphoreType.DMA,
            pltpu.SemaphoreType.DMA,
            pltpu.SemaphoreType.DMA,
        ],
    )
    def k(ids_hbm, table_hbm, out_hbm, idx_v, rows0, rows1, g0, g1, s0, s1):
        wid = lax.axis_index("s") * _NC + lax.axis_index("c")
        base = wid * _B_PER_W
        pltpu.sync_copy(ids_hbm.at[pl.ds(base, _B_PER_W)], idx_v)

        bufs = (rows0, rows1)
        gsem = (g0, g1)
        ssem = (s0, s1)

        def g_start(c, b):
            off = pl.multiple_of(c * _CHUNK, 8)
            pltpu.async_copy(
                table_hbm.at[idx_v.at[pl.ds(off, _CHUNK)]], bufs[b], gsem[b]
            )

        def g_wait(b):
            pltpu.make_async_copy(
                table_hbm.at[idx_v.at[pl.ds(0, _CHUNK)]], bufs[b], gsem[b]
            ).wait()

        def s_start(c, b):
            off = pl.multiple_of(c * _CHUNK, 8)
            pltpu.async_copy(bufs[b], out_hbm.at[pl.ds(base + off, _CHUNK)], ssem[b])

        def s_wait(b):
            pltpu.make_async_copy(
                bufs[b], out_hbm.at[pl.ds(base, _CHUNK)], ssem[b]
            ).wait()

        for b in range(_NBUF):
            g_start(b, b)

        def body(j, carry):
            for b in range(_NBUF):
                c = j * _NBUF + b
                g_wait(b)
                s_start(c, b)
                s_wait(b)
                nxt = c + _NBUF

                @pl.when(nxt < _NITER)
                def _():
                    g_start(nxt, b)

            return carry

        lax.fori_loop(0, _NITER // _NBUF, body, 0)

    return k(ids_flat, table)


def kernel(input_ids, table):
    ids_flat = input_ids.reshape(-1).astype(jnp.int32)
    out = _sc_embed(ids_flat, table)
    return out.reshape(BATCH, SEQ, D_MODEL)
